# Initial kernel scaffold; baseline (speedup 1.0000x reference)
import jax
import jax.numpy as jnp
from jax.experimental import pallas as pl


def kernel(cell_x, cell_edge_index, cell_batch, drug, gcn1_W, gcn1_b, bn1_g, bn1_b, gcn2_W, gcn2_b, bn2_g, bn2_b, cl1_W, cl1_b, bn3_g, bn3_b, cl2_W, cl2_b, d1_W, d1_b, dbn1_g, dbn1_b, d2_W, d2_b, dbn2_g, dbn2_b, f1_W, f1_b, fbn1_g, fbn1_b, f2_W, f2_b, fbn2_g, fbn2_b, f3_W, f3_b):
    raise NotImplementedError("write your pallas kernel here")


# speedup vs baseline: 11.5753x; 11.5753x over previous
import jax
import jax.numpy as jnp
from jax import lax
from jax.experimental import pallas as pl
from jax.experimental.pallas import tpu as pltpu
from jax.experimental.pallas import tpu_sc as plsc

N = 50000
E = 800000
BG = 64
F = 128
C = 8
FC = 16
NT = 16
NC = 2
NP = 50176
TPN = NP // NT
NSUB = 7
SBN = TPN // NSUB
NPASS = C // NC
EPT = E // NT
EB = 2000
NEB = EPT // EB
NSEG = 65
EPS = 1e-5


def _rsqrt(x):
    r = lax.rsqrt(x)
    return r * (1.5 - 0.5 * x * r * r)


def _dot_f32(a, b):
    ah = a.astype(jnp.bfloat16)
    al = (a - ah.astype(jnp.float32)).astype(jnp.bfloat16)
    bh = b.astype(jnp.bfloat16)
    bl = (b - bh.astype(jnp.float32)).astype(jnp.bfloat16)

    def d(u, v):
        return jnp.dot(u, v, preferred_element_type=jnp.float32)

    return d(a, b)
NEG_INF = float("-inf")
POS_INF = float("inf")

_MESH = dict(core_axis_name="c", subcore_axis_name="s", num_cores=NC,
             num_subcores=NT)
_SC_PARAMS = pltpu.CompilerParams(use_tc_tiling_on_sc=False)


def _deg_body(dst_hbm, out_hbm, acc_sh, idx_v, ones_v, buf_v):
    cid = lax.axis_index("c")
    sid = lax.axis_index("s")

    def _fill0(i, _):
        buf_v[pl.ds(i * 16, 16)] = jnp.zeros((16,), jnp.float32)
        return 0

    def _fill1(i, _):
        ones_v[pl.ds(i * 16, 16)] = jnp.ones((16,), jnp.float32)
        return 0

    lax.fori_loop(0, TPN // 16, _fill0, 0)
    lax.fori_loop(0, EB // 16, _fill1, 0)
    pltpu.sync_copy(buf_v, acc_sh.at[pl.ds(sid * TPN, TPN)])
    plsc.subcore_barrier()

    ebase = sid * EPT

    def _blk(k, _):
        pltpu.sync_copy(dst_hbm.at[pl.ds(ebase + k * EB, EB)], idx_v)
        pltpu.sync_copy(ones_v, acc_sh.at[idx_v], add=True)
        return 0

    lax.fori_loop(0, NEB, _blk, 0)
    plsc.subcore_barrier()
    pltpu.sync_copy(acc_sh.at[pl.ds(sid * TPN, TPN)],
                    out_hbm.at[cid, pl.ds(sid * TPN, TPN)])


def _deg_partials(dst):
    return pl.kernel(
        _deg_body,
        out_type=jax.ShapeDtypeStruct((NC, NP), jnp.float32),
        mesh=plsc.VectorSubcoreMesh(**_MESH),
        compiler_params=_SC_PARAMS,
        scratch_types=[
            pltpu.VMEM_SHARED((NP,), jnp.float32),
            pltpu.VMEM((EB,), jnp.int32),
            pltpu.VMEM((EB,), jnp.float32),
            pltpu.VMEM((TPN,), jnp.float32),
        ],
    )(dst)


def _accumulate_chunk(g_hbm, src_hbm, dst_hbm, acc_sh, idx_s, idx_d, rows_v,
                      gsem, sid, row_off):
    pltpu.sync_copy(g_hbm.at[pl.ds(row_off + sid * TPN, TPN), :],
                    acc_sh.at[pl.ds(sid * TPN, TPN), :])
    plsc.subcore_barrier()
    ebase = sid * EPT

    def _blk(k, _):
        pltpu.sync_copy(src_hbm.at[pl.ds(ebase + k * EB, EB)], idx_s)
        pltpu.sync_copy(dst_hbm.at[pl.ds(ebase + k * EB, EB)], idx_d)

        def _off(i, _):
            idx_s[pl.ds(i * 16, 16)] = idx_s[pl.ds(i * 16, 16)] + row_off
            return 0

        lax.fori_loop(0, EB // 16, _off, 0)
        pltpu.async_copy(g_hbm.at[idx_s], rows_v, gsem).wait()
        pltpu.sync_copy(rows_v, acc_sh.at[idx_d], add=True)
        return 0

    lax.fori_loop(0, NEB, _blk, 0)
    plsc.subcore_barrier()


def _conv1_body(g_hbm, src_hbm, dst_hbm, s_hbm,
                acc_sh, idx_s, idx_d, rows_v, gsem):
    cid = lax.axis_index("c")
    sid = lax.axis_index("s")
    for p in range(NPASS):
        chunk = cid * NPASS + p
        row_off = chunk * NP
        _accumulate_chunk(g_hbm, src_hbm, dst_hbm, acc_sh, idx_s, idx_d,
                          rows_v, gsem, sid, row_off)
        pltpu.sync_copy(acc_sh.at[pl.ds(sid * TPN, TPN), :],
                        s_hbm.at[pl.ds(row_off + sid * TPN, TPN), :])
        plsc.subcore_barrier()


def _conv1(g_flat, src, dst):
    return pl.kernel(
        _conv1_body,
        out_type=jax.ShapeDtypeStruct((C * NP, FC), jnp.float32),
        mesh=plsc.VectorSubcoreMesh(**_MESH),
        compiler_params=_SC_PARAMS,
        scratch_types=[
            pltpu.VMEM_SHARED((NP, FC), jnp.float32),
            pltpu.VMEM((EB,), jnp.int32),
            pltpu.VMEM((EB,), jnp.int32),
            pltpu.VMEM((EB, FC), jnp.float32),
            pltpu.SemaphoreType.DMA,
        ],
    )(g_flat, src, dst)


def _conv2_body(g_hbm, src_hbm, dst_hbm, dis_hbm, bias_hbm, batch_hbm,
                stats_hbm, max_hbm, min_hbm,
                acc_sh, stat_sh, max_sh, min_sh, idx_s, idx_d, rows_v, fb,
                db, bt, bb, res_v, st_all, tmax, tmin, red_v, out4, gsem):
    cid = lax.axis_index("c")
    sid = lax.axis_index("s")
    for p in range(NPASS):
        chunk = cid * NPASS + p
        row_off = chunk * NP
        pltpu.sync_copy(bias_hbm.at[pl.ds(chunk * FC, FC)], bb)

        def _init_tbl(i, _):
            tmax[i] = jnp.full((16,), NEG_INF, jnp.float32)
            tmin[i] = jnp.full((16,), POS_INF, jnp.float32)
            return 0

        lax.fori_loop(0, NSEG, _init_tbl, 0)
        _accumulate_chunk(g_hbm, src_hbm, dst_hbm, acc_sh, idx_s, idx_d,
                          rows_v, gsem, sid, row_off)
        bb0 = bb[pl.ds(0, 16)]
        nbase = sid * TPN
        z = jnp.zeros((16,), jnp.float32)

        def _sub(j, carry):
            base = nbase + j * SBN
            pltpu.sync_copy(acc_sh.at[pl.ds(base, SBN), :], fb)
            pltpu.sync_copy(dis_hbm.at[pl.ds(base, SBN)], db)
            pltpu.sync_copy(batch_hbm.at[pl.ds(base, SBN)], bt)

            def _grp(gi, car):
                s0, q0 = car
                dvec = db[pl.ds(gi * 16, 16)]
                gvec = bt[pl.ds(gi * 16, 16)]
                for r in range(16):
                    row = gi * 16 + r
                    dv = dvec[r]
                    gg = gvec[r]
                    t0 = bb0 + dv * fb[row]
                    tmax[gg] = jnp.maximum(tmax[gg], t0)
                    tmin[gg] = jnp.minimum(tmin[gg], t0)
                    s0 = s0 + t0
                    q0 = q0 + t0 * t0
                return (s0, q0)

            return lax.fori_loop(0, SBN // 16, _grp, carry)

        stats = lax.fori_loop(0, NSUB, _sub, (z, z))

        pltpu.sync_copy(tmax, max_sh.at[sid])
        pltpu.sync_copy(tmin, min_sh.at[sid])
        plsc.subcore_barrier()
        for half, sh in ((0, max_sh), (1, min_sh)):
            pltpu.sync_copy(sh.at[:, pl.ds(sid * 4, 4), :], red_v)
            for rr in range(4):
                a0 = red_v[0, rr]

                def _red(i, c0):
                    v0 = red_v[i, rr]
                    if half == 0:
                        return jnp.maximum(c0, v0)
                    return jnp.minimum(c0, v0)

                a0 = lax.fori_loop(1, NT, _red, a0)
                out4[rr] = a0
            dst_out = max_hbm if half == 0 else min_hbm
            pltpu.sync_copy(out4, dst_out.at[chunk, pl.ds(sid * 4, 4), :])

        s0, q0 = stats
        res_v[0] = s0
        res_v[1] = q0
        pltpu.sync_copy(res_v, stat_sh.at[sid])
        plsc.subcore_barrier()

        @pl.when(sid == 0)
        def _():
            pltpu.sync_copy(stat_sh, st_all)

            def _redst(i, car):
                a0, b0 = car
                return (a0 + st_all[i, 0], b0 + st_all[i, 1])

            a0, b0 = lax.fori_loop(0, NT, _redst, (z, z))
            res_v[0] = a0
            res_v[1] = b0
            pltpu.sync_copy(res_v, stats_hbm.at[chunk])

        plsc.subcore_barrier()


def _conv2(g_flat, src, dst, dis, bias, batch):
    return pl.kernel(
        _conv2_body,
        out_type=(jax.ShapeDtypeStruct((C, 2, FC), jnp.float32),
                  jax.ShapeDtypeStruct((C, BG, FC), jnp.float32),
                  jax.ShapeDtypeStruct((C, BG, FC), jnp.float32)),
        mesh=plsc.VectorSubcoreMesh(**_MESH),
        compiler_params=_SC_PARAMS,
        scratch_types=[
            pltpu.VMEM_SHARED((NP, FC), jnp.float32),
            pltpu.VMEM_SHARED((NT, 2, FC), jnp.float32),
            pltpu.VMEM_SHARED((NT, NSEG, FC), jnp.float32),
            pltpu.VMEM_SHARED((NT, NSEG, FC), jnp.float32),
            pltpu.VMEM((EB,), jnp.int32),
            pltpu.VMEM((EB,), jnp.int32),
            pltpu.VMEM((EB, FC), jnp.float32),
            pltpu.VMEM((SBN, FC), jnp.float32),
            pltpu.VMEM((SBN,), jnp.float32),
            pltpu.VMEM((SBN,), jnp.int32),
            pltpu.VMEM((FC,), jnp.float32),
            pltpu.VMEM((2, FC), jnp.float32),
            pltpu.VMEM((NT, 2, FC), jnp.float32),
            pltpu.VMEM((NSEG, FC), jnp.float32),
            pltpu.VMEM((NSEG, FC), jnp.float32),
            pltpu.VMEM((NT, 4, FC), jnp.float32),
            pltpu.VMEM((4, FC), jnp.float32),
            pltpu.SemaphoreType.DMA,
        ],
    )(g_flat, src, dst, dis, bias, batch)


def _pre1_body(xt_ref, w_ref, degp_ref, dis_ref, g_ref):
    i = pl.program_id(0)
    p = degp_ref[...]
    deg = (p[0] + p[1]) * 0.5 + 1.0
    ids = i * 1024 + lax.iota(jnp.int32, 1024)
    dis = jnp.where(ids < N, _rsqrt(deg), 0.0)
    dis_ref[...] = dis
    def bf(u):
        return u.astype(jnp.bfloat16).astype(jnp.float32)

    h = bf(xt_ref[0])[:, None] * bf(w_ref[0])[None, :]
    for k in range(1, 4):
        h = h + bf(xt_ref[k])[:, None] * bf(w_ref[k])[None, :]
    g = h * dis[:, None]
    for c in range(C):
        g_ref[c] = g[:, c * FC:(c + 1) * FC]


def _pre1(xt, w1, degp):
    bn = 1024
    return pl.pallas_call(
        _pre1_body,
        grid=(NP // bn,),
        in_specs=[
            pl.BlockSpec((4, bn), lambda i: (0, i)),
            pl.BlockSpec((4, F), lambda i: (0, 0)),
            pl.BlockSpec((NC, bn), lambda i: (0, i)),
        ],
        out_specs=[
            pl.BlockSpec((bn,), lambda i: (i,)),
            pl.BlockSpec((C, bn, FC), lambda i: (0, i, 0)),
        ],
        out_shape=[
            jax.ShapeDtypeStruct((NP,), jnp.float32),
            jax.ShapeDtypeStruct((C, NP, FC), jnp.float32),
        ],
    )(xt, w1, degp)


def _stats1_body(s_ref, dis_ref, b1_ref, out_ref):
    i = pl.program_id(0)

    @pl.when(i == 0)
    def _():
        out_ref[...] = jnp.zeros_like(out_ref)

    ids = i * 1024 + lax.iota(jnp.int32, 1024)
    mask = jnp.where(ids < N, 1.0, 0.0)
    dis = dis_ref[...] * mask
    b1 = b1_ref[...]
    for c in range(C):
        t = b1[c * FC:(c + 1) * FC][None, :] * mask[:, None] \
            + s_ref[c] * dis[:, None]
        out_ref[0, c * FC:(c + 1) * FC] += jnp.sum(t, axis=0)
        out_ref[1, c * FC:(c + 1) * FC] += jnp.sum(t * t, axis=0)


def _stats1(s1, dis, b1):
    bn = 1024
    return pl.pallas_call(
        _stats1_body,
        grid=(NP // bn,),
        in_specs=[
            pl.BlockSpec((C, bn, FC), lambda i: (0, i, 0)),
            pl.BlockSpec((bn,), lambda i: (i,)),
            pl.BlockSpec((F,), lambda i: (0,)),
        ],
        out_specs=pl.BlockSpec((2, F), lambda i: (0, 0)),
        out_shape=jax.ShapeDtypeStruct((2, F), jnp.float32),
    )(s1, dis, b1)


def _pre2_body(s_ref, stats_ref, b1_ref, g1_ref, bb1_ref, dis_ref, w2_ref,
               g2_ref):
    stats = stats_ref[...]
    b1 = b1_ref[...]
    g1 = g1_ref[...]
    bb1 = bb1_ref[...]
    dis = dis_ref[...]
    acc = None
    for c in range(C):
        sl = slice(c * FC, (c + 1) * FC)
        m = stats[0, sl] / float(N)
        v = stats[1, sl] / float(N) - m * m
        sc = g1[sl] * _rsqrt(v + EPS)
        oc = bb1[sl] + (b1[sl] - m) * sc
        t = s_ref[c] * dis[:, None]
        x1c = jnp.maximum(t * sc[None, :] + oc[None, :], 0.0)
        part = _dot_f32(x1c, w2_ref[pl.ds(c * FC, FC), :])
        acc = part if acc is None else acc + part
    g2 = acc * dis[:, None]
    for c in range(C):
        g2_ref[c] = g2[:, c * FC:(c + 1) * FC]


def _pre2(s1, stats1, b1, bn1_g, bn1_b, dis, w2):
    bn = 1024
    return pl.pallas_call(
        _pre2_body,
        grid=(NP // bn,),
        in_specs=[
            pl.BlockSpec((C, bn, FC), lambda i: (0, i, 0)),
            pl.BlockSpec((2, F), lambda i: (0, 0)),
            pl.BlockSpec((F,), lambda i: (0,)),
            pl.BlockSpec((F,), lambda i: (0,)),
            pl.BlockSpec((F,), lambda i: (0,)),
            pl.BlockSpec((bn,), lambda i: (i,)),
            pl.BlockSpec((F, F), lambda i: (0, 0)),
        ],
        out_specs=pl.BlockSpec((C, bn, FC), lambda i: (0, i, 0)),
        out_shape=jax.ShapeDtypeStruct((C, NP, FC), jnp.float32),
    )(s1, stats1, b1, bn1_g, bn1_b, dis, w2)


def _bn_exact(x, g, b):
    m = jnp.mean(x, axis=0)
    v = jnp.mean((x - m) ** 2, axis=0)
    return g * (x - m) * _rsqrt(v + EPS) + b


def _elu(x):
    return jnp.where(x > 0, x, jnp.exp(jnp.minimum(x, 0.0)) - 1.0)


def _head_body(mx_ref, mn_ref, stats_ref, b2_ref, g2_ref, bb2_ref,
               cl1w_ref, cl1b_ref, bn3g_ref, bn3b_ref, cl2w_ref, cl2b_ref,
               drug_ref, d1w_ref, d1b_ref, dbn1g_ref, dbn1b_ref,
               d2w_ref, d2b_ref, dbn2g_ref, dbn2b_ref,
               f1w_ref, f1b_ref, fbn1g_ref, fbn1b_ref,
               f2w_ref, f2b_ref, fbn2g_ref, fbn2b_ref,
               f3w_ref, f3b_ref, out_ref):
    stats = stats_ref[...]
    b2 = b2_ref[...]
    g2 = g2_ref[...]
    bb2 = bb2_ref[...]
    npad = float(NP - N)
    cols = []
    for c in range(C):
        sl = slice(c * FC, (c + 1) * FC)
        bs = b2[sl]
        m = (stats[c, 0] - npad * bs) / float(N)
        v = (stats[c, 1] - npad * bs * bs) / float(N) - m * m
        s = g2[sl] * _rsqrt(v + EPS)
        o = bb2[sl] - s * m
        picked = jnp.where(s[None, :] >= 0.0, mx_ref[c], mn_ref[c])
        cols.append(picked * s[None, :] + o[None, :])
    x3 = jnp.concatenate(cols, axis=1)
    x3 = _dot_f32(x3, cl1w_ref[...])
    x3 = x3 + cl1b_ref[...][None, :]
    x3 = jnp.maximum(_bn_exact(x3, bn3g_ref[...], bn3b_ref[...]), 0.0)
    cell = _dot_f32(x3, cl2w_ref[...])
    cell = jnp.maximum(cell + cl2b_ref[...][None, :], 0.0)

    d = _dot_f32(drug_ref[...], d1w_ref[...]) + d1b_ref[...][None, :]
    d = jnp.maximum(_bn_exact(d, dbn1g_ref[...], dbn1b_ref[...]), 0.0)
    d = _dot_f32(d, d2w_ref[...]) + d2b_ref[...][None, :]
    demb = jnp.maximum(_bn_exact(d, dbn2g_ref[...], dbn2b_ref[...]), 0.0)

    y = (_dot_f32(cell, f1w_ref[pl.ds(0, F), :])
         + _dot_f32(demb, f1w_ref[pl.ds(F, F), :])
         + f1b_ref[...][None, :])
    y = _elu(_bn_exact(y, fbn1g_ref[...], fbn1b_ref[...]))
    y = _dot_f32(y, f2w_ref[...]) + f2b_ref[...][None, :]
    y = _elu(_bn_exact(y, fbn2g_ref[...], fbn2b_ref[...]))
    y = _dot_f32(y, f3w_ref[...]) + f3b_ref[...][None, :]
    out_ref[...] = jnp.broadcast_to(y, (BG, F))


def _head(mx, mn, stats2, b2, bn2_g, bn2_b, cl1_W, cl1_b, bn3_g, bn3_b,
          cl2_W, cl2_b, drug, d1_W, d1_b, dbn1_g, dbn1_b, d2_W, d2_b,
          dbn2_g, dbn2_b, f1_W, f1_b, fbn1_g, fbn1_b, f2_W, f2_b,
          fbn2_g, fbn2_b, f3_W, f3_b):
    return pl.pallas_call(
        _head_body,
        out_shape=jax.ShapeDtypeStruct((BG, F), jnp.float32),
    )(mx, mn, stats2, b2, bn2_g, bn2_b, cl1_W, cl1_b, bn3_g, bn3_b,
      cl2_W, cl2_b, drug, d1_W, d1_b, dbn1_g, dbn1_b, d2_W, d2_b,
      dbn2_g, dbn2_b, f1_W, f1_b, fbn1_g, fbn1_b, f2_W, f2_b,
      fbn2_g, fbn2_b, f3_W, f3_b)


def kernel(cell_x, cell_edge_index, cell_batch, drug, gcn1_W, gcn1_b,
           bn1_g, bn1_b, gcn2_W, gcn2_b, bn2_g, bn2_b, cl1_W, cl1_b,
           bn3_g, bn3_b, cl2_W, cl2_b, d1_W, d1_b, dbn1_g, dbn1_b,
           d2_W, d2_b, dbn2_g, dbn2_b, f1_W, f1_b, fbn1_g, fbn1_b,
           f2_W, f2_b, fbn2_g, fbn2_b, f3_W, f3_b):
    src = cell_edge_index[0]
    dst = cell_edge_index[1]
    batch_p = jnp.pad(cell_batch, (0, NP - N), constant_values=BG)
    xt = jnp.pad(cell_x, ((0, NP - N), (0, 0))).T

    degp = _deg_partials(dst)
    dis, g1 = _pre1(xt, gcn1_W, degp)
    s1 = _conv1(g1.reshape(C * NP, FC), src, dst)
    s1 = s1.reshape(C, NP, FC)
    st1 = _stats1(s1, dis, gcn1_b)
    g2 = _pre2(s1, st1, gcn1_b, bn1_g, bn1_b, dis, gcn2_W)
    stats2, mx, mn = _conv2(g2.reshape(C * NP, FC), src, dst, dis, gcn2_b,
                            batch_p)
    y = _head(mx, mn, stats2, gcn2_b, bn2_g, bn2_b, cl1_W, cl1_b, bn3_g,
              bn3_b, cl2_W, cl2_b, drug, d1_W, d1_b, dbn1_g, dbn1_b,
              d2_W, d2_b, dbn2_g, dbn2_b, f1_W, f1_b, fbn1_g, fbn1_b,
              f2_W, f2_b, fbn2_g, fbn2_b, f3_W, f3_b)
    return y[:, 0]
